# baseline (speedup 1.0000x reference)
import functools

import jax
import jax.numpy as jnp
from jax import lax
from jax.experimental import pallas as pl
from jax.experimental.pallas import tpu as pltpu
from jax.experimental.pallas import tpu_sc as plsc

NC, NS, L = 2, 16, 16
NW = NC * NS


@functools.partial(jax.jit, static_argnames=("M", "N", "D"))
def _sc_logits(emb, plab, nlab_p, W, *, M, N, D):
    ntok = M // NW
    CT = L
    nchunks = ntok // CT
    NE = 8
    KQ = N // NE
    RQ = CT * KQ
    NSTR = RQ // 128
    H = nchunks // 2
    mesh = plsc.VectorSubcoreMesh(
        core_axis_name="c", subcore_axis_name="s", num_cores=NC, num_subcores=NS
    )

    @functools.partial(
        pl.kernel,
        out_type=[
            jax.ShapeDtypeStruct((M,), jnp.float32),
            jax.ShapeDtypeStruct((M, N), jnp.float32),
        ],
        mesh=mesh,
        compiler_params=pltpu.CompilerParams(
            needs_layout_passes=False, use_tc_tiling_on_sc=False),
        scratch_types=[
            pltpu.VMEM((CT, N), jnp.int32),
            pltpu.VMEM((CT * N,), jnp.int32),
            pltpu.VMEM((CT * N,), jnp.int32),
            pltpu.VMEM((CT,), jnp.int32),
            pltpu.VMEM((CT,), jnp.int32),
            pltpu.VMEM((CT, D), jnp.float32),
            pltpu.VMEM((CT, D), jnp.float32),
            pltpu.VMEM((RQ, D), jnp.float32),
            pltpu.VMEM((RQ, D), jnp.float32),
            pltpu.VMEM((RQ, D), jnp.float32),
            pltpu.VMEM((RQ, D), jnp.float32),
            pltpu.VMEM((CT,), jnp.float32),
            pltpu.VMEM((CT, N), jnp.float32),
            pltpu.SemaphoreType.DMA,
            pltpu.SemaphoreType.DMA,
            pltpu.SemaphoreType.DMA,
            pltpu.SemaphoreType.DMA,
            pltpu.SemaphoreType.DMA,
        ],
    )
    def body(emb_hbm, plab_hbm, nlab_hbm, w_hbm, pos_out, neg_out,
             lab_nat, lab_a, lab_b, plab_a, plab_b, embc_v, prows_v,
             buf0, buf1, buf2, buf3, pout_v, nout_v,
             sem0, sem1, sem2, sem3, sem_p):
        wid = lax.axis_index("s") * NC + lax.axis_index("c")
        base = wid * ntok
        lane = lax.iota(jnp.int32, L)
        bufs = (buf0, buf1, buf2, buf3)
        sems = (sem0, sem1, sem2, sem3)
        rotk = [(lane + kp) & (L - 1) for kp in range(KQ)]
        rowk = [rotk[kp] * CT + lane for kp in range(KQ)]
        rotkCT = [rotk[kp] * CT for kp in range(KQ)]
        laneN = lane * N

        def dcols(d):
            s = d & (L - 1)
            return ((lane + s) & (L - 1)) + (d - s)

        def issue_eighth(lab_v, e, b):
            for s in range(NSTR):
                pltpu.async_copy(
                    w_hbm.at[lab_v.at[pl.ds(e * RQ + s * 128, 128)]],
                    bufs[b].at[pl.ds(s * 128, 128)], sems[b])

        def wait_eighth(lab_v, e, b):
            for s in range(NSTR):
                pltpu.make_async_copy(
                    w_hbm.at[lab_v.at[pl.ds(e * RQ + s * 128, 128)]],
                    bufs[b].at[pl.ds(s * 128, 128)], sems[b]).wait()

        def stage_chunk(lab_v, plab_v, ci):
            row0 = base + ci * CT
            pltpu.sync_copy(nlab_hbm.at[pl.ds(row0, CT)], lab_nat)
            pltpu.sync_copy(plab_hbm.at[pl.ds(row0, CT)], plab_v)
            pltpu.async_copy(w_hbm.at[plab_v], prows_v, sem_p)

            def ebody(e, _):
                eKQ = e * KQ
                ebase_w = e * (KQ * CT) + lane
                for kp in range(KQ):
                    v = plsc.load_gather(lab_nat, [lane, eKQ + rotk[kp]])
                    plsc.store_scatter(lab_v, [ebase_w + rotkCT[kp]], v)
                return _

            lax.fori_loop(0, NE, ebody, 0)

        def head_compute(plab_v, ci):
            row0 = base + ci * CT
            pltpu.sync_copy(emb_hbm.at[pl.ds(row0, CT)], embc_v)
            pltpu.make_async_copy(w_hbm.at[plab_v], prows_v, sem_p).wait()

            def pbody(d, pacc):
                cold = dcols(d)
                pv = plsc.load_gather(prows_v, [lane, cold])
                ev = plsc.load_gather(embc_v, [lane, cold])
                return pacc + pv * ev

            pout_v[...] = lax.fori_loop(0, D, pbody,
                                        jnp.zeros((L,), jnp.float32))

        def compute_eighth(e, b):
            buf = bufs[b]

            def dbody(d, accs):
                cold = dcols(d)
                ev = plsc.load_gather(embc_v, [lane, cold])
                return tuple(
                    accs[kp] + plsc.load_gather(buf, [rowk[kp], cold]) * ev
                    for kp in range(KQ))

            accs = lax.fori_loop(
                0, D, dbody,
                tuple(jnp.zeros((L,), jnp.float32) for _ in range(KQ)),
                unroll=2)
            eKQ = e * KQ
            for kp in range(KQ):
                plsc.store_scatter(nout_v, [lane, eKQ + rotk[kp]], accs[kp])

        def tail(ci):
            row0 = base + ci * CT
            pltpu.sync_copy(nout_v, neg_out.at[pl.ds(row0, CT)])
            pltpu.sync_copy(pout_v, pos_out.at[pl.ds(row0, CT)])

        def half_chunk_steps(lab_v, plab_v, lab_n, plab_n, ci, ci_next,
                             do_next):
            for e in range(NE):
                b = e % 4
                if e < NE - 2:
                    issue_eighth(lab_v, e + 2, (e + 2) % 4)
                else:
                    def _prep(e=e):
                        if e == NE - 2:
                            stage_chunk(lab_n, plab_n, ci_next)
                        issue_eighth(lab_n, e + 2 - NE, (e + 2) % 4)
                    if do_next is None:
                        _prep()
                    else:
                        pl.when(do_next)(_prep)
                if e == 0:
                    head_compute(plab_v, ci)
                wait_eighth(lab_v, e, b)
                compute_eighth(e, b)
            tail(ci)

        stage_chunk(lab_a, plab_a, 0)
        issue_eighth(lab_a, 0, 0)
        issue_eighth(lab_a, 1, 1)

        def hbody(h, _):
            cA = 2 * h
            half_chunk_steps(lab_a, plab_a, lab_b, plab_b, cA, cA + 1, None)
            half_chunk_steps(lab_b, plab_b, lab_a, plab_a, cA + 1, cA + 2,
                             h < H - 1)
            return _

        lax.fori_loop(0, H, hbody, 0)

    return body(emb, plab, nlab_p, W)


def kernel(model_embeddings, positive_labels, negative_labels,
           target_padding_mask, W):
    B, S, D = model_embeddings.shape
    N = negative_labels.shape[-1]
    M = B * S
    ntok = M // NW
    CT = L
    nchunks = ntok // CT
    NE = 8
    KQ = N // NE
    emb = model_embeddings.reshape(M, D)
    plab = positive_labels.reshape(M).astype(jnp.int32)
    nlab2 = negative_labels.reshape(M, N).astype(jnp.int32)
    pos_logits, neg_logits = _sc_logits(emb, plab, nlab2, W, M=M, N=N, D=D)
    return (pos_logits.reshape(M, 1), neg_logits,
            positive_labels.reshape(M, 1), negative_labels.reshape(M, N))


# speedup vs baseline: 1.1350x; 1.1350x over previous
import functools

import jax
import jax.numpy as jnp
from jax import lax
from jax.experimental import pallas as pl
from jax.experimental.pallas import tpu as pltpu
from jax.experimental.pallas import tpu_sc as plsc

NC, NS, L = 2, 16, 16
NW = NC * NS


@functools.partial(jax.jit, static_argnames=("M", "N", "D"))
def _sc_logits(emb, plab, nlab_p, W, *, M, N, D):
    ntok = M // NW
    CT = L
    nchunks = ntok // CT
    NE = 8
    KQ = N // NE
    RQ = CT * KQ
    NSTR = RQ // 128
    H = nchunks // 2
    mesh = plsc.VectorSubcoreMesh(
        core_axis_name="c", subcore_axis_name="s", num_cores=NC, num_subcores=NS
    )

    @functools.partial(
        pl.kernel,
        out_type=[
            jax.ShapeDtypeStruct((M,), jnp.float32),
            jax.ShapeDtypeStruct((M, N), jnp.float32),
        ],
        mesh=mesh,
        compiler_params=pltpu.CompilerParams(
            needs_layout_passes=False, use_tc_tiling_on_sc=False),
        scratch_types=[
            pltpu.VMEM((CT, N), jnp.int32),
            pltpu.VMEM((CT * N,), jnp.int32),
            pltpu.VMEM((CT * N,), jnp.int32),
            pltpu.VMEM((CT,), jnp.int32),
            pltpu.VMEM((CT,), jnp.int32),
            pltpu.VMEM((CT, D), jnp.float32),
            pltpu.VMEM((CT, D), jnp.float32),
            pltpu.VMEM((RQ, D), jnp.float32),
            pltpu.VMEM((RQ, D), jnp.float32),
            pltpu.VMEM((RQ, D), jnp.float32),
            pltpu.VMEM((RQ, D), jnp.float32),
            pltpu.VMEM((CT,), jnp.float32),
            pltpu.VMEM((CT, N), jnp.float32),
            pltpu.SemaphoreType.DMA,
            pltpu.SemaphoreType.DMA,
            pltpu.SemaphoreType.DMA,
            pltpu.SemaphoreType.DMA,
            pltpu.SemaphoreType.DMA,
        ],
    )
    def body(emb_hbm, plab_hbm, nlab_hbm, w_hbm, pos_out, neg_out,
             lab_nat, lab_a, lab_b, plab_a, plab_b, embc_v, prows_v,
             buf0, buf1, buf2, buf3, pout_v, nout_v,
             sem0, sem1, sem2, sem3, sem_p):
        wid = lax.axis_index("s") * NC + lax.axis_index("c")
        base = wid * ntok
        lane = lax.iota(jnp.int32, L)
        bufs = (buf0, buf1, buf2, buf3)
        sems = (sem0, sem1, sem2, sem3)
        rotk = [(lane + kp) & (L - 1) for kp in range(KQ)]
        rowk = [rotk[kp] * CT + lane for kp in range(KQ)]
        rotkCT = [rotk[kp] * CT for kp in range(KQ)]
        laneN = lane * N

        def dcols(d):
            s = d & (L - 1)
            return ((lane + s) & (L - 1)) + (d - s)

        def issue_eighth(lab_v, e, b):
            for s in range(NSTR):
                pltpu.async_copy(
                    w_hbm.at[lab_v.at[pl.ds(e * RQ + s * 128, 128)]],
                    bufs[b].at[pl.ds(s * 128, 128)], sems[b])

        def wait_eighth(lab_v, e, b):
            for s in range(NSTR):
                pltpu.make_async_copy(
                    w_hbm.at[lab_v.at[pl.ds(e * RQ + s * 128, 128)]],
                    bufs[b].at[pl.ds(s * 128, 128)], sems[b]).wait()

        def stage_chunk(lab_v, plab_v, ci):
            row0 = base + ci * CT
            pltpu.sync_copy(nlab_hbm.at[pl.ds(row0, CT)], lab_nat)
            pltpu.sync_copy(plab_hbm.at[pl.ds(row0, CT)], plab_v)
            pltpu.async_copy(w_hbm.at[plab_v], prows_v, sem_p)

            def ebody(e, _):
                eKQ = e * KQ
                ebase_w = e * (KQ * CT) + lane
                for kp in range(KQ):
                    v = plsc.load_gather(lab_nat, [lane, eKQ + rotk[kp]])
                    plsc.store_scatter(lab_v, [ebase_w + rotkCT[kp]], v)
                return _

            lax.fori_loop(0, NE, ebody, 0)

        def head_compute(plab_v, ci):
            row0 = base + ci * CT
            pltpu.sync_copy(emb_hbm.at[pl.ds(row0, CT)], embc_v)
            pltpu.make_async_copy(w_hbm.at[plab_v], prows_v, sem_p).wait()

            def pbody(d, pacc):
                cold = dcols(d)
                pv = plsc.load_gather(prows_v, [lane, cold])
                ev = plsc.load_gather(embc_v, [lane, cold])
                return pacc + pv * ev

            pout_v[...] = lax.fori_loop(0, D, pbody,
                                        jnp.zeros((L,), jnp.float32))

        def compute_eighth(e, b):
            buf = bufs[b]

            def dbody(d, accs):
                cold = dcols(d)
                ev = plsc.load_gather(embc_v, [lane, cold])
                return tuple(
                    accs[kp] + plsc.load_gather(buf, [rowk[kp], cold]) * ev
                    for kp in range(KQ))

            accs = lax.fori_loop(
                0, D, dbody,
                tuple(jnp.zeros((L,), jnp.float32) for _ in range(KQ)))
            eKQ = e * KQ
            for kp in range(KQ):
                plsc.store_scatter(nout_v, [lane, eKQ + rotk[kp]], accs[kp])

        def tail(ci):
            row0 = base + ci * CT
            pltpu.sync_copy(nout_v, neg_out.at[pl.ds(row0, CT)])
            pltpu.sync_copy(pout_v, pos_out.at[pl.ds(row0, CT)])

        def half_chunk_steps(lab_v, plab_v, lab_n, plab_n, ci, ci_next,
                             do_next):
            for e in range(NE):
                b = e % 4
                if e < NE - 3:
                    issue_eighth(lab_v, e + 3, (e + 3) % 4)
                else:
                    def _prep(e=e):
                        if e == NE - 3:
                            stage_chunk(lab_n, plab_n, ci_next)
                        issue_eighth(lab_n, e + 3 - NE, (e + 3) % 4)
                    if do_next is None:
                        _prep()
                    else:
                        pl.when(do_next)(_prep)
                if e == 0:
                    head_compute(plab_v, ci)
                wait_eighth(lab_v, e, b)
                compute_eighth(e, b)
            tail(ci)

        stage_chunk(lab_a, plab_a, 0)
        issue_eighth(lab_a, 0, 0)
        issue_eighth(lab_a, 1, 1)
        issue_eighth(lab_a, 2, 2)

        def hbody(h, _):
            cA = 2 * h
            half_chunk_steps(lab_a, plab_a, lab_b, plab_b, cA, cA + 1, None)
            half_chunk_steps(lab_b, plab_b, lab_a, plab_a, cA + 1, cA + 2,
                             h < H - 1)
            return _

        lax.fori_loop(0, H, hbody, 0)

    return body(emb, plab, nlab_p, W)


def kernel(model_embeddings, positive_labels, negative_labels,
           target_padding_mask, W):
    B, S, D = model_embeddings.shape
    N = negative_labels.shape[-1]
    M = B * S
    ntok = M // NW
    CT = L
    nchunks = ntok // CT
    NE = 8
    KQ = N // NE
    emb = model_embeddings.reshape(M, D)
    plab = positive_labels.reshape(M).astype(jnp.int32)
    nlab2 = negative_labels.reshape(M, N).astype(jnp.int32)
    pos_logits, neg_logits = _sc_logits(emb, plab, nlab2, W, M=M, N=N, D=D)
    return (pos_logits.reshape(M, 1), neg_logits,
            positive_labels.reshape(M, 1), negative_labels.reshape(M, N))
